# Initial kernel scaffold; baseline (speedup 1.0000x reference)
import jax
import jax.numpy as jnp
from jax.experimental import pallas as pl


def kernel(pos, x, orientation, w_emb, tp_w, fc_w1, fc_w2, wd0, wd1):
    raise NotImplementedError("write your pallas kernel here")


# speedup vs baseline: 6.1245x; 6.1245x over previous
import functools
import jax
import jax.numpy as jnp
from jax.experimental import pallas as pl

N = 10000
D_INF = 128
MUL = 32
MAX_RADIUS = 0.073
NUM_BASIS = 10
E_MAX = 262144
BLK = 2048
STEP = MAX_RADIUS / 11.0
CEMB = 1.14136 * 2.718281828459045 ** 2
N2 = 1.0 / (MUL * 2.0) ** 0.5
N3 = 1.0 / (MUL * 3.0) ** 0.5
SSC = 1.0 / 15.0 ** 0.5
SQ3 = 3.0 ** 0.5
I_SQ3 = 1.0 / 3.0 ** 0.5
I_SQ2 = 1.0 / 2.0 ** 0.5

_DIMS = {0: (64, 128), 1: (352, 224), 2: (320, 128)}


def _edge_body(layer, t_ref, g_ref, fc1_ref, fc2_ref, o_ref):
    t = t_ref[...]
    g = g_ref[...]
    evx, evy, evz = g[:, 0:1], g[:, 1:2], g[:, 2:3]
    valid = g[:, 3:4]
    el2 = evx * evx + evy * evy + evz * evz
    el = jnp.sqrt(el2)
    inv = jnp.where(el > 0.0, 1.0 / jnp.where(el > 0.0, el, 1.0), 0.0)
    shx, shy, shz = SQ3 * evx * inv, SQ3 * evy * inv, SQ3 * evz * inv
    lane = jax.lax.broadcasted_iota(
        jnp.int32, (el.shape[0], 16), 1).astype(jnp.float32)
    diff = (el - (lane + 1.0) * STEP) * (1.0 / STEP)
    dd = diff * diff
    inside = jnp.logical_and(dd < 1.0, lane < 10.0)
    safe = jnp.where(inside, dd, 0.0)
    emb = jnp.where(inside, CEMB * jnp.exp(-1.0 / (1.0 - safe)), 0.0)
    hid = jnp.dot(emb, fc1_ref[...], preferred_element_type=jnp.float32)
    hid = hid * jax.nn.sigmoid(hid)
    rr = jnp.sum(hid * fc2_ref[0:1, :], axis=1, keepdims=True) * 0.125 * valid
    r2 = rr * N2
    r3 = rr * N3
    if layer == 0:
        m0e = t[:, 0:32] * r2
        t3 = t[:, 32:64] * r3
        out = jnp.concatenate([m0e, t3 * shx, t3 * shy, t3 * shz], axis=1)
    elif layer == 1:
        dot5 = (t[:, 160:192] * shx + t[:, 192:224] * shy
                + t[:, 224:256] * shz) * I_SQ3
        m0e = (t[:, 0:32] + dot5) * r2
        u3 = t[:, 32:64]
        m1ox = (u3 * shx + t[:, 64:96]) * r3
        m1oy = (u3 * shy + t[:, 96:128]) * r3
        m1oz = (u3 * shz + t[:, 128:160]) * r3
        u6x, u6y, u6z = t[:, 256:288], t[:, 288:320], t[:, 320:352]
        m1ex = (u6y * shz - u6z * shy) * (I_SQ2 * 1.0) * r3
        m1ey = (u6z * shx - u6x * shz) * I_SQ2 * r3
        m1ez = (u6x * shy - u6y * shx) * I_SQ2 * r3
        out = jnp.concatenate([m0e, m1ox, m1oy, m1oz, m1ex, m1ey, m1ez],
                              axis=1)
    else:
        v5x, v5y, v5z = t[:, 32:64], t[:, 64:96], t[:, 96:128]
        dot5 = (v5x * shx + v5y * shy + v5z * shz) * I_SQ3
        m0e = (t[:, 0:32] + dot5) * r2
        v6x, v6y, v6z = t[:, 128:160], t[:, 160:192], t[:, 192:224]
        m1ex = (t[:, 224:256] + (v6y * shz - v6z * shy) * I_SQ2) * r3
        m1ey = (t[:, 256:288] + (v6z * shx - v6x * shz) * I_SQ2) * r3
        m1ez = (t[:, 288:320] + (v6x * shy - v6y * shx) * I_SQ2) * r3
        out = jnp.concatenate([m0e, m1ex, m1ey, m1ez], axis=1)
    o_ref[...] = out


@functools.lru_cache(maxsize=None)
def _edge_kernel(layer):
    d_in, d_out = _DIMS[layer]
    return pl.pallas_call(
        functools.partial(_edge_body, layer),
        grid=(E_MAX // BLK,),
        in_specs=[
            pl.BlockSpec((BLK, d_in), lambda i: (i, 0)),
            pl.BlockSpec((BLK, 8), lambda i: (i, 0)),
            pl.BlockSpec((16, 64), lambda i: (0, 0)),
            pl.BlockSpec((8, 64), lambda i: (0, 0)),
        ],
        out_specs=pl.BlockSpec((BLK, d_out), lambda i: (i, 0)),
        out_shape=jax.ShapeDtypeStruct((E_MAX, d_out), jnp.float32),
    )


def _mulmat(h, w):
    return jnp.einsum('ncu,uv->ncv', h, w).reshape(N, 96)


def kernel(pos, x, orientation, w_emb, tp_w, fc_w1, fc_w2, wd0, wd1):
    sq = jnp.sum(pos * pos, axis=1)
    d2 = sq[:, None] + sq[None, :] - 2.0 * (pos @ pos.T)
    mask = d2 <= MAX_RADIUS ** 2
    count = jnp.sum(mask)
    row, col = jnp.nonzero(mask, size=E_MAX, fill_value=0)
    ev = pos[col] - pos[row]
    in_range = jnp.arange(E_MAX) < count
    valid = jnp.logical_and(in_range, jnp.sum(ev * ev, axis=1) > 0.0)
    G = jnp.concatenate(
        [ev, valid.astype(jnp.float32)[:, None],
         jnp.zeros((E_MAX, 4), jnp.float32)], axis=1)
    row_s = jnp.where(in_range, row, N).astype(jnp.int32)
    col = col.astype(jnp.int32)

    fc1p = jnp.zeros((3, 16, 64), jnp.float32).at[:, :10, :].set(fc_w1)
    fc2p = jnp.zeros((3, 8, 64), jnp.float32).at[:, 0, :].set(fc_w2[:, :, 0])

    h0e = x @ w_emb * (1.0 / float(D_INF) ** 0.5)
    seg = functools.partial(jax.ops.segment_sum, num_segments=N,
                            indices_are_sorted=True)

    W = tp_w[0]
    T = jnp.concatenate([h0e @ W[2], h0e @ W[3]], axis=1)
    M = _edge_kernel(0)(jnp.take(T, col, axis=0), G, fc1p[0], fc2p[0])
    d = seg(M, row_s)
    h0e = h0e + d[:, :32] * SSC
    h1o = d[:, 32:].reshape(N, 3, MUL) * SSC

    W = tp_w[1]
    T = jnp.concatenate(
        [h0e @ W[2], h0e @ W[3], _mulmat(h1o, W[4]), _mulmat(h1o, W[5]),
         _mulmat(h1o, W[6])], axis=1)
    M = _edge_kernel(1)(jnp.take(T, col, axis=0), G, fc1p[1], fc2p[1])
    d = seg(M, row_s)
    h0e = h0e + d[:, :32] * SSC
    h1o = h1o + d[:, 32:128].reshape(N, 3, MUL) * SSC
    h1e = d[:, 128:].reshape(N, 3, MUL) * SSC

    W = tp_w[2]
    T = jnp.concatenate(
        [h0e @ W[2], _mulmat(h1o, W[5]), _mulmat(h1o, W[6]),
         _mulmat(h1e, W[7])], axis=1)
    M = _edge_kernel(2)(jnp.take(T, col, axis=0), G, fc1p[2], fc2p[2])
    d = seg(M, row_s)
    h0e = h0e + d[:, :32] * SSC
    h1e = h1e + d[:, 32:].reshape(N, 3, MUL) * SSC

    c0 = jnp.mean(h0e, axis=0) @ wd0 * (1.0 / float(MUL) ** 0.5)
    c1 = jnp.mean(h1e, axis=0) @ wd1 * (1.0 / float(MUL) ** 0.5)
    sh_coeffs = jnp.concatenate([c0[None], c1])[None, :]
    theta, phi = orientation[..., 0], orientation[..., 1]
    v = jnp.stack([jnp.sin(theta) * jnp.cos(phi),
                   jnp.sin(theta) * jnp.sin(phi),
                   jnp.cos(theta)], axis=-1)
    sh_q = jnp.concatenate([jnp.ones_like(theta)[..., None], SQ3 * v],
                           axis=-1)
    return jnp.sum(sh_coeffs * sh_q, axis=-1)


# speedup vs baseline: 7.5297x; 1.2294x over previous
import functools
import jax
import jax.numpy as jnp
from jax import lax
from jax.experimental import pallas as pl
from jax.experimental.pallas import tpu as pltpu
from jax.experimental.pallas import tpu_sc as plsc

N = 10000
D_INF = 128
MUL = 32
MAX_RADIUS = 0.073
E_MAX = 262144
BLK = 2048
STEP = MAX_RADIUS / 11.0
CEMB = 1.14136 * 2.718281828459045 ** 2
N2 = 1.0 / (MUL * 2.0) ** 0.5
N3 = 1.0 / (MUL * 3.0) ** 0.5
SSC = 1.0 / 15.0 ** 0.5
SQ3 = 3.0 ** 0.5
I_SQ3 = 1.0 / 3.0 ** 0.5
I_SQ2 = 1.0 / 2.0 ** 0.5

NW = 32
RPW = 313
NPAD = NW * RPW
K = 64

_DIMS = {0: (64, 128), 1: (352, 224), 2: (320, 128)}
_DPAD = {0: 128, 1: 384, 2: 384}


def _geom_body(g_ref, fc1_ref, fc2_ref, o_ref):
    g = g_ref[...]
    evx, evy, evz = g[:, 0:1], g[:, 1:2], g[:, 2:3]
    valid = g[:, 3:4]
    el2 = evx * evx + evy * evy + evz * evz
    el = jnp.sqrt(el2)
    inv = jnp.where(el > 0.0, 1.0 / jnp.where(el > 0.0, el, 1.0), 0.0)
    shx, shy, shz = SQ3 * evx * inv, SQ3 * evy * inv, SQ3 * evz * inv
    lane = jax.lax.broadcasted_iota(
        jnp.int32, (el.shape[0], 16), 1).astype(jnp.float32)
    diff = (el - (lane + 1.0) * STEP) * (1.0 / STEP)
    dd = diff * diff
    inside = jnp.logical_and(dd < 1.0, lane < 10.0)
    safe = jnp.where(inside, dd, 0.0)
    emb = jnp.where(inside, CEMB * jnp.exp(-1.0 / (1.0 - safe)), 0.0)
    rs = []
    for l in range(3):
        hid = jnp.dot(emb, fc1_ref[16 * l:16 * (l + 1), :],
                      preferred_element_type=jnp.float32)
        hid = hid * jax.nn.sigmoid(hid)
        rs.append(jnp.sum(hid * fc2_ref[8 * l:8 * l + 1, :], axis=1,
                          keepdims=True) * 0.125 * valid)
    o_ref[...] = jnp.concatenate(
        [shx, shy, shz, rs[0], rs[1], rs[2],
         jnp.zeros_like(shx), jnp.zeros_like(shx)], axis=1)


@functools.lru_cache(maxsize=None)
def _geom_kernel():
    return pl.pallas_call(
        _geom_body,
        grid=(E_MAX // BLK,),
        in_specs=[
            pl.BlockSpec((BLK, 8), lambda i: (i, 0)),
            pl.BlockSpec((48, 64), lambda i: (0, 0)),
            pl.BlockSpec((24, 64), lambda i: (0, 0)),
        ],
        out_specs=pl.BlockSpec((BLK, 8), lambda i: (i, 0)),
        out_shape=jax.ShapeDtypeStruct((E_MAX, 8), jnp.float32),
    )


def _sc_layer_body(layer, t_hbm, col_hbm, row_hbm, g_hbm, b_hbm, out_hbm,
                   b_v, idx_v, rows_v, g_v, row_v, acc_v, sem):
    d_in, f_out = _DIMS[layer]
    info = plsc.get_sparse_core_info()
    nc = info.num_cores
    wid = lax.axis_index("s") * nc + lax.axis_index("c")
    pltpu.sync_copy(b_hbm, b_v)
    row0 = wid * RPW
    bb = b_v[pl.ds(2 * wid, 16)]

    zeros16 = jnp.zeros((16,), jnp.float32)

    def zbody(i, _):
        acc_v[pl.ds(i * 16, 16)] = zeros16
        return 0

    lax.fori_loop(0, RPW * f_out // 16, zbody, 0)

    e0 = bb[0]
    e1 = bb[1]
    nchunk = (e1 - e0 + (K - 1)) // K

    def chunk(c, _):
        ea = pl.multiple_of(e0 + c * K, 8)
        pltpu.sync_copy(col_hbm.at[pl.ds(ea, K)], idx_v)
        pltpu.sync_copy(row_hbm.at[pl.ds(ea, K)], row_v.at[pl.ds(0, K)])
        pltpu.sync_copy(g_hbm.at[pl.ds(ea * 8, K * 8)], g_v.at[pl.ds(0, K * 8)])
        pltpu.async_copy(t_hbm.at[idx_v], rows_v, sem).wait()

        def edge(j, _):
            ge = ea + j
            lr = row_v[pl.ds(j, 16)][0] - row0
            ok = jnp.logical_and(
                ge < e1,
                jnp.logical_and(lr >= 0, lr < RPW))

            @pl.when(ok)
            def _():
                gv = g_v[pl.ds(j * 8, 16)]
                shx = gv[0]
                shy = gv[1]
                shz = gv[2]
                rr = gv[3 + layer]
                r2 = rr * N2
                r3 = rr * N3
                ab = lr * f_out

                def ld(t, h):
                    return rows_v[j, pl.ds(32 * t + 16 * h, 16)]

                def st(t, h, v):
                    plsc.addupdate(acc_v.at[pl.ds(ab + 32 * t + 16 * h, 16)], v)

                for h in range(2):
                    if layer == 0:
                        t2, t3 = ld(0, h), ld(1, h)
                        st(0, h, t2 * r2)
                        st(1, h, t3 * (shx * r3))
                        st(2, h, t3 * (shy * r3))
                        st(3, h, t3 * (shz * r3))
                    elif layer == 1:
                        u2, u3 = ld(0, h), ld(1, h)
                        u4x, u4y, u4z = ld(2, h), ld(3, h), ld(4, h)
                        u5x, u5y, u5z = ld(5, h), ld(6, h), ld(7, h)
                        u6x, u6y, u6z = ld(8, h), ld(9, h), ld(10, h)
                        dot5 = (u5x * shx + u5y * shy + u5z * shz) * I_SQ3
                        st(0, h, (u2 + dot5) * r2)
                        st(1, h, (u3 * shx + u4x) * r3)
                        st(2, h, (u3 * shy + u4y) * r3)
                        st(3, h, (u3 * shz + u4z) * r3)
                        cs = I_SQ2 * r3
                        st(4, h, (u6y * shz - u6z * shy) * cs)
                        st(5, h, (u6z * shx - u6x * shz) * cs)
                        st(6, h, (u6x * shy - u6y * shx) * cs)
                    else:
                        v2 = ld(0, h)
                        v5x, v5y, v5z = ld(1, h), ld(2, h), ld(3, h)
                        v6x, v6y, v6z = ld(4, h), ld(5, h), ld(6, h)
                        v7x, v7y, v7z = ld(7, h), ld(8, h), ld(9, h)
                        dot5 = (v5x * shx + v5y * shy + v5z * shz) * I_SQ3
                        st(0, h, (v2 + dot5) * r2)
                        st(1, h, (v7x + (v6y * shz - v6z * shy) * I_SQ2) * r3)
                        st(2, h, (v7y + (v6z * shx - v6x * shz) * I_SQ2) * r3)
                        st(3, h, (v7z + (v6x * shy - v6y * shx) * I_SQ2) * r3)
            return 0

        lax.fori_loop(0, K, edge, 0)
        return 0

    lax.fori_loop(0, nchunk, chunk, 0)
    pltpu.sync_copy(acc_v, out_hbm.at[pl.ds(wid * RPW * f_out, RPW * f_out)])


@functools.lru_cache(maxsize=None)
def _sc_layer_kernel(layer):
    d_in, f_out = _DIMS[layer]
    mesh = plsc.VectorSubcoreMesh(core_axis_name="c", subcore_axis_name="s")
    return pl.kernel(
        functools.partial(_sc_layer_body, layer),
        mesh=mesh,
        out_type=jax.ShapeDtypeStruct((NPAD * f_out,), jnp.float32),
        scratch_types=[
            pltpu.VMEM((2 * NW + 16,), jnp.int32),
            pltpu.VMEM((K,), jnp.int32),
            pltpu.VMEM((K, _DPAD[layer]), jnp.float32),
            pltpu.VMEM((K * 8 + 16,), jnp.float32),
            pltpu.VMEM((K + 16,), jnp.int32),
            pltpu.VMEM((RPW * f_out,), jnp.float32),
            pltpu.SemaphoreType.DMA,
        ],
    )


def _mulmat(h, w):
    return jnp.einsum('ncu,uv->ncv', h, w).reshape(N, 96)


def kernel(pos, x, orientation, w_emb, tp_w, fc_w1, fc_w2, wd0, wd1):
    sq = jnp.sum(pos * pos, axis=1)
    d2 = sq[:, None] + sq[None, :] - 2.0 * (pos @ pos.T)
    mask = d2 <= MAX_RADIUS ** 2
    count = jnp.sum(mask)
    row, col = jnp.nonzero(mask, size=E_MAX, fill_value=0)
    ev = pos[col] - pos[row]
    in_range = jnp.arange(E_MAX) < count
    valid = jnp.logical_and(in_range, jnp.sum(ev * ev, axis=1) > 0.0)
    G = jnp.concatenate(
        [ev, valid.astype(jnp.float32)[:, None],
         jnp.zeros((E_MAX, 4), jnp.float32)], axis=1)
    row_s = jnp.where(in_range, row, N).astype(jnp.int32)
    col = col.astype(jnp.int32)

    bnd_rows = jnp.minimum(jnp.arange(33, dtype=jnp.int32) * RPW, N)
    eb = jnp.searchsorted(row_s, bnd_rows, side='left').astype(jnp.int32)
    starts = (eb[:32] // 8) * 8
    bounds = jnp.concatenate(
        [jnp.stack([starts, eb[1:]], axis=1).reshape(64),
         jnp.zeros((16,), jnp.int32)])

    fc1p = jnp.zeros((3, 16, 64), jnp.float32).at[:, :10, :].set(fc_w1)
    fc2p = jnp.zeros((3, 8, 64), jnp.float32).at[:, 0, :].set(fc_w2[:, :, 0])
    G2 = _geom_kernel()(G, fc1p.reshape(48, 64), fc2p.reshape(24, 64))

    h0e = x @ w_emb * (1.0 / float(D_INF) ** 0.5)

    def run_layer(l, T):
        Tp = jnp.concatenate(
            [T, jnp.zeros((N, _DPAD[l] - T.shape[1]), jnp.float32)], axis=1)
        flat = _sc_layer_kernel(l)(Tp, col, row_s, G2.reshape(-1), bounds)
        return flat.reshape(NPAD, _DIMS[l][1])[:N] * SSC

    W = tp_w[0]
    d = run_layer(0, jnp.concatenate([h0e @ W[2], h0e @ W[3]], axis=1))
    h0e = h0e + d[:, :32]
    h1o = d[:, 32:].reshape(N, 3, MUL)

    W = tp_w[1]
    d = run_layer(1, jnp.concatenate(
        [h0e @ W[2], h0e @ W[3], _mulmat(h1o, W[4]), _mulmat(h1o, W[5]),
         _mulmat(h1o, W[6])], axis=1))
    h0e = h0e + d[:, :32]
    h1o = h1o + d[:, 32:128].reshape(N, 3, MUL)
    h1e = d[:, 128:].reshape(N, 3, MUL)

    W = tp_w[2]
    d = run_layer(2, jnp.concatenate(
        [h0e @ W[2], _mulmat(h1o, W[5]), _mulmat(h1o, W[6]),
         _mulmat(h1e, W[7])], axis=1))
    h0e = h0e + d[:, :32]
    h1e = h1e + d[:, 32:].reshape(N, 3, MUL)

    c0 = jnp.mean(h0e, axis=0) @ wd0 * (1.0 / float(MUL) ** 0.5)
    c1 = jnp.mean(h1e, axis=0) @ wd1 * (1.0 / float(MUL) ** 0.5)
    sh_coeffs = jnp.concatenate([c0[None], c1])[None, :]
    theta, phi = orientation[..., 0], orientation[..., 1]
    v = jnp.stack([jnp.sin(theta) * jnp.cos(phi),
                   jnp.sin(theta) * jnp.sin(phi),
                   jnp.cos(theta)], axis=-1)
    sh_q = jnp.concatenate([jnp.ones_like(theta)[..., None], SQ3 * v],
                           axis=-1)
    return jnp.sum(sh_coeffs * sh_q, axis=-1)
